# baseline (speedup 1.0000x reference)
import functools

import jax
import jax.numpy as jnp
from jax import lax
from jax.experimental import pallas as pl
from jax.experimental.pallas import tpu as pltpu
from jax.experimental.pallas import tpu_sc as plsc

VOCAB = 1000000
EMB = 64
BATCH = 16384
HIST = 50

_NW = 32
_BATCH_PER_W = BATCH // _NW
_G = 8
_NSTEPS = _BATCH_PER_W // _G
_PADEMB = 128

assert _BATCH_PER_W % _G == 0 and _NSTEPS % 2 == 0

_mesh = plsc.VectorSubcoreMesh(core_axis_name="c", subcore_axis_name="s")


@functools.partial(
    pl.kernel,
    out_type=jax.ShapeDtypeStruct((BATCH, HIST, EMB), jnp.float32),
    mesh=_mesh,
    scratch_types=[
        pltpu.VMEM((_G, HIST), jnp.int32),
        pltpu.VMEM((_G, HIST), jnp.int32),
        pltpu.VMEM((_G, HIST, _PADEMB), jnp.float32),
        pltpu.VMEM((_G, HIST, _PADEMB), jnp.float32),
        pltpu.SemaphoreType.DMA,
        pltpu.SemaphoreType.DMA,
        pltpu.SemaphoreType.DMA,
        pltpu.SemaphoreType.DMA,
        pltpu.SemaphoreType.DMA,
        pltpu.SemaphoreType.DMA,
    ],
    compiler_params=pltpu.CompilerParams(use_tc_tiling_on_sc=False),
)
def _gather_kernel(table_hbm, x_hbm, out_hbm,
                   idx0, idx1, rows0, rows1,
                   si0, si1, sg0, sg1, ss0, ss1):
    wid = lax.axis_index("s") * 2 + lax.axis_index("c")
    base_batch = wid * _BATCH_PER_W

    def idx_load(c, idxv, sem):
        b0 = base_batch + c * _G
        pltpu.async_copy(x_hbm.at[pl.ds(b0, _G)], idxv, sem)

    def wait_idx(idxv, sem):
        pltpu.make_async_copy(x_hbm.at[pl.ds(0, _G)], idxv, sem).wait()

    def gathers(idxv, rowsv, sem):
        for i in range(_G):
            pltpu.async_copy(table_hbm.at[idxv.at[i]], rowsv.at[i], sem)

    def wait_gathers(rowsv, sem):
        for i in range(_G):
            pltpu.make_async_copy(table_hbm.at[idx0.at[0]], rowsv.at[i], sem).wait()

    def store(c, rowsv, sem):
        b0 = base_batch + c * _G
        pltpu.async_copy(rowsv.at[:, :, pl.ds(0, EMB)],
                         out_hbm.at[pl.ds(b0, _G)], sem)

    def wait_store(rowsv, sem):
        pltpu.make_async_copy(rowsv.at[:, :, pl.ds(0, EMB)],
                              out_hbm.at[pl.ds(0, _G)], sem).wait()

    idx_load(0, idx0, si0)
    idx_load(1, idx1, si1)

    def body(j, _):
        c0 = 2 * j
        c1 = c0 + 1

        wait_idx(idx0, si0)

        @pl.when(j > 0)
        def _():
            wait_store(rows0, ss0)

        gathers(idx0, rows0, sg0)

        wait_idx(idx1, si1)

        @pl.when(j > 0)
        def _():
            wait_store(rows1, ss1)

        gathers(idx1, rows1, sg1)

        wait_gathers(rows0, sg0)
        store(c0, rows0, ss0)

        @pl.when(c0 + 2 < _NSTEPS)
        def _():
            idx_load(c0 + 2, idx0, si0)

        wait_gathers(rows1, sg1)
        store(c1, rows1, ss1)

        @pl.when(c1 + 2 < _NSTEPS)
        def _():
            idx_load(c1 + 2, idx1, si1)

        return 0

    lax.fori_loop(0, _NSTEPS // 2, body, 0)

    wait_store(rows0, ss0)
    wait_store(rows1, ss1)


@jax.jit
def kernel(x, table):
    tpad = jnp.pad(table, ((0, 0), (0, _PADEMB - EMB)))
    return _gather_kernel(tpad, x.astype(jnp.int32))


# speedup vs baseline: 1.0547x; 1.0547x over previous
import functools

import jax
import jax.numpy as jnp
from jax import lax
from jax.experimental import pallas as pl
from jax.experimental.pallas import tpu as pltpu
from jax.experimental.pallas import tpu_sc as plsc

VOCAB = 1000000
EMB = 64
BATCH = 16384
HIST = 50

_NW = 32
_BATCH_PER_W = BATCH // _NW
_G = 8
_NSLOT = 4
_NSTEPS = _BATCH_PER_W // _G

assert _BATCH_PER_W % _G == 0 and _NSTEPS % _NSLOT == 0

_mesh = plsc.VectorSubcoreMesh(core_axis_name="c", subcore_axis_name="s")

_scratch = (
    [pltpu.VMEM((_G, HIST), jnp.int32) for _ in range(_NSLOT)]
    + [pltpu.VMEM((_G, HIST, EMB), jnp.float32) for _ in range(_NSLOT)]
    + [pltpu.SemaphoreType.DMA for _ in range(3 * _NSLOT)]
)


@functools.partial(
    pl.kernel,
    out_type=jax.ShapeDtypeStruct((BATCH, HIST, EMB), jnp.float32),
    mesh=_mesh,
    scratch_types=_scratch,
    compiler_params=pltpu.CompilerParams(use_tc_tiling_on_sc=False),
)
def _gather_kernel(table_hbm, x_hbm, out_hbm, *refs):
    idx = refs[:_NSLOT]
    rows = refs[_NSLOT:2 * _NSLOT]
    si = refs[2 * _NSLOT:3 * _NSLOT]
    sg = refs[3 * _NSLOT:4 * _NSLOT]
    ss = refs[4 * _NSLOT:5 * _NSLOT]

    wid = lax.axis_index("s") * 2 + lax.axis_index("c")
    base_batch = wid * _BATCH_PER_W

    def idx_load(c, s):
        b0 = base_batch + c * _G
        pltpu.async_copy(x_hbm.at[pl.ds(b0, _G)], idx[s], si[s])

    def wait_idx(s):
        pltpu.make_async_copy(x_hbm.at[pl.ds(0, _G)], idx[s], si[s]).wait()

    def gathers(s):
        for i in range(_G):
            pltpu.async_copy(table_hbm.at[idx[s].at[i]], rows[s].at[i], sg[s])

    def wait_gathers(s):
        for i in range(_G):
            pltpu.make_async_copy(table_hbm.at[idx[s].at[0]], rows[s].at[i],
                                  sg[s]).wait()

    def store(c, s):
        b0 = base_batch + c * _G
        pltpu.async_copy(rows[s], out_hbm.at[pl.ds(b0, _G)], ss[s])

    def wait_store(s):
        pltpu.make_async_copy(rows[s], out_hbm.at[pl.ds(0, _G)], ss[s]).wait()

    for s in range(_NSLOT):
        idx_load(s, s)

    def body(j, _):
        c_base = _NSLOT * j

        for s in range(_NSLOT):
            wait_idx(s)

            @pl.when(j > 0)
            def _(s=s):
                wait_store(s)

            gathers(s)

        for s in range(_NSLOT):
            c = c_base + s
            wait_gathers(s)
            store(c, s)

            @pl.when(c + _NSLOT < _NSTEPS)
            def _(c=c, s=s):
                idx_load(c + _NSLOT, s)

        return 0

    lax.fori_loop(0, _NSTEPS // _NSLOT, body, 0)

    for s in range(_NSLOT):
        wait_store(s)


@jax.jit
def kernel(x, table):
    return _gather_kernel(table, x.astype(jnp.int32))
